# baseline (speedup 1.0000x reference)
import functools

import jax
import jax.numpy as jnp
import numpy as np
from jax import lax
from jax.experimental import pallas as pl
from jax.experimental.pallas import tpu as pltpu
from jax.experimental.pallas import tpu_sc as plsc

_ALPHA = 0.1
_B, _D = 16384, 2048
_NC, _NS = 2, 16
_NW = _NC * _NS
_RPW = _B // _NW
_C = 8
_NCHUNK = _RPW // _C
_NBUF = 2
_NI = _NCHUNK // _NBUF
_L = 16


def _sc_mixup(x, perm, alpha_rep):
    mesh = plsc.VectorSubcoreMesh(core_axis_name="c", subcore_axis_name="s",
                                  num_cores=_NC, num_subcores=_NS)

    @functools.partial(
        pl.kernel,
        out_type=jax.ShapeDtypeStruct((_B, _D), jnp.float32),
        mesh=mesh,
        scratch_types=[
            pltpu.VMEM((_RPW,), jnp.int32),
            pltpu.VMEM((_NCHUNK, _C * _L), jnp.float32),
            pltpu.VMEM((_NBUF, _C, _D), jnp.float32),
            pltpu.VMEM((_NBUF, _C, _D), jnp.float32),
            pltpu.VMEM((_NBUF, _C, _D), jnp.float32),
            pltpu.SemaphoreType.DMA,
            pltpu.SemaphoreType.DMA,
            pltpu.SemaphoreType.DMA,
            pltpu.SemaphoreType.DMA,
            pltpu.SemaphoreType.DMA,
            pltpu.SemaphoreType.DMA,
        ],
    )
    def k(x_hbm, perm_hbm, alpha_hbm, out_hbm, idx_v, al_v, g_v, own_v, o_v,
          sg0, sg1, so0, so1, su0, su1):
        sem_g = [sg0, sg1]
        sem_own = [so0, so1]
        sem_out = [su0, su1]
        wid = lax.axis_index("s") * _NC + lax.axis_index("c")
        base = wid * _RPW

        def fill(b, g):
            pltpu.async_copy(x_hbm.at[idx_v.at[pl.ds(g * _C, _C)]],
                             g_v.at[b], sem_g[b])
            pltpu.async_copy(x_hbm.at[pl.ds(base + g * _C, _C)],
                             own_v.at[b], sem_own[b])

        for b in range(_NBUF):
            pltpu.async_copy(x_hbm.at[pl.ds(base + b * _C, _C)],
                             own_v.at[b], sem_own[b])
        pltpu.sync_copy(perm_hbm.at[pl.ds(base, _RPW)], idx_v)
        alget = pltpu.async_copy(alpha_hbm.at[pl.ds(wid * _NCHUNK, _NCHUNK)],
                                 al_v, su0)
        for b in range(_NBUF):
            pltpu.async_copy(x_hbm.at[idx_v.at[pl.ds(b * _C, _C)]],
                             g_v.at[b], sem_g[b])
        alget.wait()

        def outer(i, carry):
            g0 = i * _NBUF
            for b in range(_NBUF):
                g = g0 + b
                row0 = base + g * _C

                @pl.when(i > 0)
                def _():
                    pltpu.make_async_copy(
                        o_v.at[b], out_hbm.at[pl.ds(row0 - _NBUF * _C, _C)],
                        sem_out[b]).wait()

                pltpu.make_async_copy(x_hbm.at[idx_v.at[pl.ds(g * _C, _C)]],
                                      g_v.at[b], sem_g[b]).wait()
                pltpu.make_async_copy(x_hbm.at[pl.ds(row0, _C)], own_v.at[b],
                                      sem_own[b]).wait()

                for r in range(_C):
                    a = al_v[g, pl.ds(r * _L, _L)]
                    na = 1.0 - a

                    @plsc.parallel_loop(0, _D // _L, unroll=8)
                    def _(j, b=b, r=r, a=a, na=na):
                        sl = pl.ds(j * _L, _L)
                        o_v[b, r, sl] = na * own_v[b, r, sl] + a * g_v[b, r, sl]

                pltpu.async_copy(o_v.at[b], out_hbm.at[pl.ds(row0, _C)],
                                 sem_out[b])

                @pl.when(i < _NI - 1)
                def _():
                    fill(b, g + _NBUF)
            return carry

        lax.fori_loop(0, _NI, outer, 0)

        for b in range(_NBUF):
            row0 = base + (_NCHUNK - _NBUF + b) * _C
            pltpu.make_async_copy(o_v.at[b], out_hbm.at[pl.ds(row0, _C)],
                                  sem_out[b]).wait()

    return k(x, perm, alpha_rep)


def _mixup_params():
    with jax.default_device(jax.local_devices(backend="cpu")[0]):
        kp = jax.random.fold_in(jax.random.key(0), 1)
        ka = jax.random.fold_in(jax.random.key(0), 2)
        perm = jax.random.permutation(kp, _B)
        beta = jax.random.beta(ka, _ALPHA, _ALPHA, (_B,)).astype(jnp.float32)
        alpha = jnp.minimum(beta, 1.0 - beta)
        alpha_rep = np.repeat(np.asarray(alpha, np.float32), _L)
        return (np.asarray(perm, dtype=np.int32),
                alpha_rep.reshape(_B // _C, _C * _L))


_PERM_NP, _ALPHA_REP_NP = _mixup_params()


def kernel(input):
    x = input.astype(jnp.float32)
    perm = jnp.asarray(_PERM_NP)
    alpha_rep = jnp.asarray(_ALPHA_REP_NP)
    return _sc_mixup(x, perm, alpha_rep)
